# Initial kernel scaffold; baseline (speedup 1.0000x reference)
import jax
import jax.numpy as jnp
from jax.experimental import pallas as pl


def kernel(adjacency_matrix, arrivals, departures, is_hard_to_match, active_agents, timestep, total_timesteps, emb_W1, emb_b1, emb_W2, emb_b2, g1_W, g1_as, g1_ad, g1_b, n1_w, n1_b, g2_W, g2_as, g2_ad, g2_b, n2_w, n2_b, g3_W, g3_as, g3_ad, g3_b, ff_W1, ff_b1, ff_W2, ff_b2):
    raise NotImplementedError("write your pallas kernel here")


# speedup vs baseline: 20.0245x; 20.0245x over previous
import functools

import jax
import jax.numpy as jnp
from jax import lax
from jax.experimental import pallas as pl
from jax.experimental.pallas import tpu as pltpu

N = 10000
NP = 10240
H = 128
BD = 512
BS = 512
DGRID = NP // BD
SGRID = NP // BS
NEG = -1e30


def _prep_body(scal_ref, arr_ref, dep_ref, hard_ref, act_ref,
               w1_ref, w2_ref, b1_ref, b2_ref, g1w_ref,
               xT_ref, h1T_ref):
    j = pl.program_id(0)
    cb = arr_ref.shape[1]
    col = j * cb + lax.broadcasted_iota(jnp.int32, (1, cb), 1)
    inb = col < N
    ts = scal_ref[0, 0]
    tss = scal_ref[1, 0]
    m2 = scal_ref[2, 0]
    m4 = scal_ref[3, 0]
    m7 = scal_ref[4, 0]
    arr = arr_ref[...]
    dep = dep_ref[...]
    tsa = jnp.where(inb, (ts - arr) / (dep - arr), 0.0)
    hard = jnp.where(inb, hard_ref[...], 0.0)
    aac = jnp.sum(act_ref[...]) / N
    ones = jnp.ones((1, cb), jnp.float32)
    nfT = jnp.concatenate(
        [tsa, tss * ones, m2 * ones, m4 * ones, m7 * ones, hard, aac * ones],
        axis=0)
    wt = jnp.dot(w2_ref[...], w1_ref[...],
                 preferred_element_type=jnp.float32)
    beff = jnp.dot(w2_ref[...], b1_ref[...],
                   preferred_element_type=jnp.float32) + b2_ref[...]
    xT = jnp.dot(wt, nfT, preferred_element_type=jnp.float32) + beff
    xT_ref[...] = xT
    h1T_ref[...] = jnp.dot(g1w_ref[...], xT,
                           preferred_element_type=jnp.float32)


def _prep(scal, arrivals, departures, hard, active, w1, w2, b1c, b2c, g1w):
    CB = 2048
    grid = (NP // CB,)
    return pl.pallas_call(
        _prep_body,
        grid=grid,
        in_specs=[
            pl.BlockSpec(memory_space=pltpu.SMEM),
            pl.BlockSpec((1, CB), lambda j: (0, j)),
            pl.BlockSpec((1, CB), lambda j: (0, j)),
            pl.BlockSpec((1, CB), lambda j: (0, j)),
            pl.BlockSpec((1, N), lambda j: (0, 0)),
            pl.BlockSpec((H, 7), lambda j: (0, 0)),
            pl.BlockSpec((H, H), lambda j: (0, 0)),
            pl.BlockSpec((H, 1), lambda j: (0, 0)),
            pl.BlockSpec((H, 1), lambda j: (0, 0)),
            pl.BlockSpec((2 * H, H), lambda j: (0, 0)),
        ],
        out_specs=[
            pl.BlockSpec((H, CB), lambda j: (0, j)),
            pl.BlockSpec((2 * H, CB), lambda j: (0, j)),
        ],
        out_shape=[
            jax.ShapeDtypeStruct((H, NP), jnp.float32),
            jax.ShapeDtypeStruct((2 * H, NP), jnp.float32),
        ],
    )(scal, arrivals, departures, hard, active, w1, w2, b1c, b2c, g1w)


def _flash_body(F, a_ref, hs_ref, hd_ref, gas_ref, gad_ref, bias_ref,
                out_ref, stats_ref, m_ref, den_ref, acc_ref):
    d = pl.program_id(0)
    s = pl.program_id(1)

    @pl.when(s == 0)
    def _():
        m_ref[...] = jnp.full((1, BD), NEG, jnp.float32)
        den_ref[...] = jnp.zeros((1, BD), jnp.float32)
        acc_ref[...] = jnp.zeros((F, BD), jnp.float32)

    row = s * BS + lax.broadcasted_iota(jnp.int32, (BS, 1), 0)
    col = d * BD + lax.broadcasted_iota(jnp.int32, (1, BD), 1)
    inb = (row < N) & (col < N)
    diag = (row == col) & (row < N)

    hs = hs_ref[...]
    a_s = lax.dot_general(hs, gas_ref[...],
                          (((0,), (0,)), ((), ())),
                          preferred_element_type=jnp.float32)
    a_d = lax.dot_general(gad_ref[...], hd_ref[...],
                          (((0,), (0,)), ((), ())),
                          preferred_element_type=jnp.float32)
    e = a_s + a_d
    e = jnp.where(e >= 0.0, e, 0.2 * e)
    c = jnp.where(inb, a_ref[...].astype(jnp.float32), 0.0)
    c = c + diag.astype(jnp.float32)
    em = jnp.where(c > 0.0, e, NEG)

    m_old = m_ref[...]
    m_new = jnp.maximum(m_old, jnp.max(em, axis=0, keepdims=True))
    scale = jnp.exp(m_old - m_new)
    p = jnp.exp(em - m_new) * c
    den = den_ref[...] * scale + jnp.sum(p, axis=0, keepdims=True)
    acc = acc_ref[...] * scale + lax.dot_general(
        hs, p, (((1,), (0,)), ((), ())),
        preferred_element_type=jnp.float32)
    m_ref[...] = m_new
    den_ref[...] = den
    acc_ref[...] = acc

    @pl.when(s == SGRID - 1)
    def _():
        outb = acc / (den + 1e-16) + bias_ref[...]
        out_ref[...] = outb
        cmask = (col < N).astype(jnp.float32)
        bsum = jnp.sum(outb * cmask)
        bsq = jnp.sum(outb * outb * cmask)
        vals = jnp.concatenate([jnp.full((1, 128), bsum, jnp.float32),
                                jnp.full((1, 128), bsq, jnp.float32)], axis=0)

        @pl.when(d == 0)
        def _():
            stats_ref[...] = vals

        @pl.when(d > 0)
        def _():
            stats_ref[...] = stats_ref[...] + vals


def _flash(A, hT, gas_col, gad_col, bias_col, F):
    return pl.pallas_call(
        functools.partial(_flash_body, F),
        grid=(DGRID, SGRID),
        in_specs=[
            pl.BlockSpec((BS, BD), lambda d, s: (s, d)),
            pl.BlockSpec((F, BS), lambda d, s: (0, s)),
            pl.BlockSpec((F, BD), lambda d, s: (0, d)),
            pl.BlockSpec((F, 1), lambda d, s: (0, 0)),
            pl.BlockSpec((F, 1), lambda d, s: (0, 0)),
            pl.BlockSpec((F, 1), lambda d, s: (0, 0)),
        ],
        out_specs=[
            pl.BlockSpec((F, BD), lambda d, s: (0, d)),
            pl.BlockSpec((2, 128), lambda d, s: (0, 0)),
        ],
        out_shape=[
            jax.ShapeDtypeStruct((F, NP), jnp.float32),
            jax.ShapeDtypeStruct((2, 128), jnp.float32),
        ],
        scratch_shapes=[
            pltpu.VMEM((1, BD), jnp.float32),
            pltpu.VMEM((1, BD), jnp.float32),
            pltpu.VMEM((F, BD), jnp.float32),
        ],
        compiler_params=pltpu.CompilerParams(
            dimension_semantics=("arbitrary", "arbitrary")),
    )(A, hT, hT, gas_col, gad_col, bias_col)


def _mid_body(out_ref, ms_ref, nw_ref, nb_ref, gw_ref, hn_ref):
    mean = ms_ref[0, 0]
    istd = ms_ref[1, 0]
    r = (out_ref[...] - mean) * istd * nw_ref[...] + nb_ref[...]
    r = jnp.maximum(r, 0.0)
    hn_ref[...] = jnp.dot(gw_ref[...], r, preferred_element_type=jnp.float32)


def _mid(outT, mean_istd, nw_col, nb_col, gw, F, F2):
    CB = 2048
    return pl.pallas_call(
        _mid_body,
        grid=(NP // CB,),
        in_specs=[
            pl.BlockSpec((F, CB), lambda j: (0, j)),
            pl.BlockSpec(memory_space=pltpu.SMEM),
            pl.BlockSpec((F, 1), lambda j: (0, 0)),
            pl.BlockSpec((F, 1), lambda j: (0, 0)),
            pl.BlockSpec((F2, F), lambda j: (0, 0)),
        ],
        out_specs=pl.BlockSpec((F2, CB), lambda j: (0, j)),
        out_shape=jax.ShapeDtypeStruct((F2, NP), jnp.float32),
    )(outT, mean_istd, nw_col, nb_col, gw)


def _final_body(xT_ref, h3_ref, act_ref, w1_ref, b1_ref, w2_ref, b2_ref,
                out_ref):
    x1 = xT_ref[...] + h3_ref[...]
    y1 = jnp.dot(w1_ref[...], x1, preferred_element_type=jnp.float32)
    x2 = x1 + jnp.maximum(y1 + b1_ref[...], 0.0)
    y2 = jnp.dot(w2_ref[...], x2, preferred_element_type=jnp.float32)
    x3 = x2 + jnp.maximum(y2 + b2_ref[...], 0.0)
    x3 = x3 * act_ref[...]
    out_ref[...] = x3.T


def _final(xT, h3T, active, ffw1, ffb1c, ffw2, ffb2c):
    CB = 512
    return pl.pallas_call(
        _final_body,
        grid=(NP // CB,),
        in_specs=[
            pl.BlockSpec((H, CB), lambda j: (0, j)),
            pl.BlockSpec((H, CB), lambda j: (0, j)),
            pl.BlockSpec((1, CB), lambda j: (0, j)),
            pl.BlockSpec((H, H), lambda j: (0, 0)),
            pl.BlockSpec((H, 1), lambda j: (0, 0)),
            pl.BlockSpec((H, H), lambda j: (0, 0)),
            pl.BlockSpec((H, 1), lambda j: (0, 0)),
        ],
        out_specs=pl.BlockSpec((CB, H), lambda j: (j, 0)),
        out_shape=jax.ShapeDtypeStruct((N, H), jnp.float32),
    )(xT, h3T, active, ffw1, ffb1c, ffw2, ffb2c)


def kernel(adjacency_matrix, arrivals, departures, is_hard_to_match,
           active_agents, timestep, total_timesteps,
           emb_W1, emb_b1, emb_W2, emb_b2,
           g1_W, g1_as, g1_ad, g1_b, n1_w, n1_b,
           g2_W, g2_as, g2_ad, g2_b, n2_w, n2_b,
           g3_W, g3_as, g3_ad, g3_b,
           ff_W1, ff_b1, ff_W2, ff_b2):
    f32 = jnp.float32
    A = adjacency_matrix[0]

    ts = timestep[0]
    tss = ts / total_timesteps[0]
    scal = jnp.stack([ts, tss, tss % 2.0, tss % 4.0, tss % 7.0]
                     ).reshape(5, 1).astype(f32)

    col = lambda v: v.reshape(-1, 1).astype(f32)

    xT, h1T = _prep(scal, arrivals, departures, is_hard_to_match,
                    active_agents, emb_W1, emb_W2, col(emb_b1), col(emb_b2),
                    g1_W)

    def layer(hT, gas, gad, gb, F):
        return _flash(A, hT, col(gas), col(gad), col(gb), F)

    def ln_stats(stats, F):
        cnt = jnp.float32(N * F)
        mean = stats[0, 0] / cnt
        var = jnp.maximum(stats[1, 0] / cnt - mean * mean, 0.0)
        istd = 1.0 / (jnp.sqrt(var) + 1e-5)
        return jnp.stack([mean, istd]).reshape(2, 1)

    o1T, st1 = layer(h1T, g1_as, g1_ad, g1_b, 2 * H)
    h2T = _mid(o1T, ln_stats(st1, 2 * H), col(n1_w), col(n1_b), g2_W,
               2 * H, 2 * H)
    o2T, st2 = layer(h2T, g2_as, g2_ad, g2_b, 2 * H)
    h3T = _mid(o2T, ln_stats(st2, 2 * H), col(n2_w), col(n2_b), g3_W,
               2 * H, H)
    o3T, _ = layer(h3T, g3_as, g3_ad, g3_b, H)

    xout = _final(xT, o3T, active_agents, ff_W1, col(ff_b1), ff_W2,
                  col(ff_b2))
    return (xout.reshape(1, N, H), active_agents)
